# Initial kernel scaffold; baseline (speedup 1.0000x reference)
import jax
import jax.numpy as jnp
from jax.experimental import pallas as pl


def kernel(x):
    raise NotImplementedError("write your pallas kernel here")


# speedup vs baseline: 4.2217x; 4.2217x over previous
import jax
import jax.numpy as jnp
from jax.experimental import pallas as pl

_ROWS = 16384
_COLS = 128
_BLK = 2048


def _body(x_ref, o_ref):
    blk = x_ref[...]
    b = blk[:, 124:125]
    cvb = blk[:, 125:126]
    cvl = blk[:, 126:127]
    qg = blk[:, 127:128]
    c = b / (1.0 / cvb - 1.0)
    d = c / cvl - c
    col = jax.lax.broadcasted_iota(jnp.int32, blk.shape, 1)
    res = jnp.where(
        col < 124,
        blk,
        jnp.where(
            col == 124,
            b,
            jnp.where(col == 125, c, jnp.where(col == 126, (1.0 - qg) * d, qg * d)),
        ),
    )
    o_ref[...] = res


def kernel(x):
    grid = (_ROWS // _BLK,)
    return pl.pallas_call(
        _body,
        grid=grid,
        in_specs=[pl.BlockSpec((_BLK, _COLS), lambda i: (i, 0))],
        out_specs=pl.BlockSpec((_BLK, _COLS), lambda i: (i, 0)),
        out_shape=jax.ShapeDtypeStruct((_ROWS, _COLS), jnp.float32),
    )(x)
